# Initial kernel scaffold; baseline (speedup 1.0000x reference)
import jax
import jax.numpy as jnp
from jax.experimental import pallas as pl


def kernel(sparse_0, sparse_1, sparse_2, sparse_3, sparse_4, sparse_5, sparse_6, sparse_7, sparse_8, sparse_9, sparse_10, sparse_11, sparse_12, sparse_13, sparse_14, sparse_15, sparse_16, sparse_17, sparse_18, sparse_19, sparse_20, sparse_21, sparse_22, sparse_23, sparse_24, sparse_25, dense_0, tables, W1, b1, W2, b2, Wout, bout):
    raise NotImplementedError("write your pallas kernel here")


# speedup vs baseline: 7.4179x; 7.4179x over previous
import functools

import jax
import jax.numpy as jnp
from jax.experimental import pallas as pl
from jax.experimental.pallas import tpu as pltpu
from jax.experimental.pallas import tpu_sc as plsc

_N_SPARSE = 26
_VOCAB = 100000
_EMB = 16
_B = 16384
_DENSE = 13
_H1, _H2, _OUT = 256, 128, 64
_EMB_IN = _N_SPARSE * _EMB
_GID = _N_SPARSE * _B
_WIN = 128
_BM = 1024


def _sc_gather(tab_flat, idx_flat):
    mesh = plsc.VectorSubcoreMesh(core_axis_name="core", subcore_axis_name="subcore")

    @functools.partial(
        pl.kernel,
        out_type=jax.ShapeDtypeStruct((_GID, _EMB), jnp.float32),
        mesh=mesh,
        compiler_params=pltpu.CompilerParams(use_tc_tiling_on_sc=False),
    )
    def k(tab_hbm, idx_hbm, o_hbm):
        def body(i_vmem, o_vmem):
            pltpu.sync_copy(tab_hbm.at[i_vmem.at[0]], o_vmem)

        pltpu.emit_pipeline(
            body,
            grid=(_GID // _WIN,),
            in_specs=[pl.BlockSpec((1, _WIN), index_map=lambda i: (0, i))],
            out_specs=[pl.BlockSpec((_WIN, _EMB), index_map=lambda i: (i, 0))],
            core_axis_name=("core", "subcore"),
            dimension_semantics=(pltpu.PARALLEL,),
        )(idx_hbm, o_hbm)

    return k(tab_flat, idx_flat)


def _mlp_body(emb_ref, dense_ref, w1a_ref, w1b_ref, b1_ref, w2_ref, b2_ref,
              wo_ref, bo_ref, o_ref):
    dn = (((1,), (0,)), ((), ()))
    f32 = jnp.float32
    h = jax.lax.dot_general(emb_ref[...], w1a_ref[...], dn,
                            preferred_element_type=f32)
    h = h + jax.lax.dot_general(dense_ref[...], w1b_ref[...], dn,
                                preferred_element_type=f32)
    h = jnp.maximum(h + b1_ref[...], 0.0)
    h = jax.lax.dot_general(h, w2_ref[...], dn, preferred_element_type=f32)
    h = jnp.maximum(h + b2_ref[...], 0.0)
    out = jax.lax.dot_general(h, wo_ref[...], dn, preferred_element_type=f32)
    out = out + bo_ref[...]
    ssq = jnp.sum(out * out, axis=1, keepdims=True)
    denom = jnp.maximum(jnp.sqrt(ssq), 1e-12)
    o_ref[...] = out / denom


def _tc_mlp(emb, dense_0, W1, b1, W2, b2, Wout, bout):
    w1a = W1[:_EMB_IN]
    w1b = W1[_EMB_IN:]
    full = lambda shape: pl.BlockSpec(shape, lambda i: (0, 0))
    return pl.pallas_call(
        _mlp_body,
        grid=(_B // _BM,),
        in_specs=[
            pl.BlockSpec((_BM, _EMB_IN), lambda i: (i, 0)),
            pl.BlockSpec((_BM, _DENSE), lambda i: (i, 0)),
            full((_EMB_IN, _H1)),
            full((_DENSE, _H1)),
            full((1, _H1)),
            full((_H1, _H2)),
            full((1, _H2)),
            full((_H2, _OUT)),
            full((1, _OUT)),
        ],
        out_specs=pl.BlockSpec((_BM, _OUT), lambda i: (i, 0)),
        out_shape=jax.ShapeDtypeStruct((_B, _OUT), jnp.float32),
    )(emb, dense_0, w1a, w1b, b1[None, :], W2, b2[None, :], Wout, bout[None, :])


def kernel(sparse_0, sparse_1, sparse_2, sparse_3, sparse_4, sparse_5,
           sparse_6, sparse_7, sparse_8, sparse_9, sparse_10, sparse_11,
           sparse_12, sparse_13, sparse_14, sparse_15, sparse_16, sparse_17,
           sparse_18, sparse_19, sparse_20, sparse_21, sparse_22, sparse_23,
           sparse_24, sparse_25, dense_0, tables, W1, b1, W2, b2, Wout, bout):
    sparse = [sparse_0, sparse_1, sparse_2, sparse_3, sparse_4, sparse_5,
              sparse_6, sparse_7, sparse_8, sparse_9, sparse_10, sparse_11,
              sparse_12, sparse_13, sparse_14, sparse_15, sparse_16,
              sparse_17, sparse_18, sparse_19, sparse_20, sparse_21,
              sparse_22, sparse_23, sparse_24, sparse_25]
    idx = jnp.stack(sparse, axis=1)
    offs = (jnp.arange(_N_SPARSE, dtype=jnp.int32) * _VOCAB)[None, :]
    idx_flat = (idx + offs).reshape(1, _GID)
    tab_flat = tables.reshape(_N_SPARSE * _VOCAB, _EMB)
    emb = _sc_gather(tab_flat, idx_flat).reshape(_B, _EMB_IN)
    return _tc_mlp(emb, dense_0, W1, b1, W2, b2, Wout, bout)
